# Initial kernel scaffold; baseline (speedup 1.0000x reference)
import jax
import jax.numpy as jnp
from jax.experimental import pallas as pl


def kernel(cast_idx, crew_idx, cast_table, crew_table, W1, b1, W2, b2):
    raise NotImplementedError("write your pallas kernel here")


# speedup vs baseline: 6.9950x; 6.9950x over previous
import functools

import jax
import jax.numpy as jnp
from jax import lax
from jax.experimental import pallas as pl
from jax.experimental.pallas import tpu as pltpu
from jax.experimental.pallas import tpu_sc as plsc

B = 16384
L = 20
NUM_CAST = 1000
NUM_CREW = 1000
EMB = 32
HID = 128

NC = 2
NS = 16
LANES = 16
NW = NC * NS
BPW = B // NW
GROUPS = BPW // LANES


@functools.cache
def _make_pool_kernel(interpret=False):
    mesh = plsc.VectorSubcoreMesh(
        core_axis_name="c", subcore_axis_name="s",
        num_cores=NC, num_subcores=NS)

    @functools.partial(
        pl.kernel,
        out_type=jax.ShapeDtypeStruct((NW, 2 * EMB, BPW), jnp.float32),
        mesh=mesh,
        scratch_types=[
            pltpu.VMEM((NUM_CAST * EMB,), jnp.float32),
            pltpu.VMEM((NUM_CREW * EMB,), jnp.float32),
            pltpu.VMEM((BPW * L,), jnp.int32),
            pltpu.VMEM((BPW * L,), jnp.int32),
            pltpu.VMEM((2 * EMB, BPW), jnp.float32),
        ],
        compiler_params=pltpu.CompilerParams(needs_layout_passes=False),
        interpret=interpret,
    )
    def pool(cast_tab_hbm, crew_tab_hbm, cidx_hbm, kidx_hbm, out_hbm,
             cast_v, crew_v, cidx_v, kidx_v, out_v):
        wid = lax.axis_index("s") * NC + lax.axis_index("c")
        pltpu.sync_copy(cast_tab_hbm, cast_v)
        pltpu.sync_copy(crew_tab_hbm, crew_v)
        pltpu.sync_copy(cidx_hbm.at[wid], cidx_v)
        pltpu.sync_copy(kidx_hbm.at[wid], kidx_v)

        def group(g, carry):
            rowbase = (lax.iota(jnp.int32, LANES) + g * LANES) * L
            for idx_v, tab_v, off in ((cidx_v, cast_v, 0),
                                      (kidx_v, crew_v, EMB)):
                bases = [plsc.load_gather(idx_v, [rowbase + l]) * EMB
                         for l in range(L)]
                for d in range(EMB):
                    acc = plsc.load_gather(tab_v, [bases[0] + d])
                    for l in range(1, L):
                        acc = acc + plsc.load_gather(tab_v, [bases[l] + d])
                    out_v[off + d, pl.ds(g * LANES, LANES)] = acc * (1.0 / L)
            return carry

        lax.fori_loop(0, GROUPS, group, 0)
        pltpu.sync_copy(out_v, out_hbm.at[wid])

    return pool


def _mlp_body(x_ref, w1_ref, b1_ref, w2_ref, b2_ref, o_ref):
    x = x_ref[0]
    h = lax.dot_general(w1_ref[...], x, (((0,), (0,)), ((), ())),
                        preferred_element_type=jnp.float32)
    h = jnp.maximum(h + b1_ref[...][:, None], 0.0)
    o = lax.dot_general(w2_ref[...], h, (((0,), (0,)), ((), ())),
                        preferred_element_type=jnp.float32)
    o_ref[...] = (o + b2_ref[...][:, None])[None]


@functools.cache
def _make_mlp_call(interpret=False):
    return pl.pallas_call(
        _mlp_body,
        grid=(NW,),
        in_specs=[
            pl.BlockSpec((1, 2 * EMB, BPW), lambda i: (i, 0, 0)),
            pl.BlockSpec((2 * EMB, HID), lambda i: (0, 0)),
            pl.BlockSpec((HID,), lambda i: (0,)),
            pl.BlockSpec((HID, 1), lambda i: (0, 0)),
            pl.BlockSpec((1,), lambda i: (0,)),
        ],
        out_specs=pl.BlockSpec((1, 1, BPW), lambda i: (i, 0, 0)),
        out_shape=jax.ShapeDtypeStruct((NW, 1, BPW), jnp.float32),
        interpret=interpret,
    )


def kernel(cast_idx, crew_idx, cast_table, crew_table, W1, b1, W2, b2):
    cidx = cast_idx.astype(jnp.int32).reshape(NW, BPW * L)
    kidx = crew_idx.astype(jnp.int32).reshape(NW, BPW * L)
    pooled = _make_pool_kernel()(cast_table.reshape(-1),
                                 crew_table.reshape(-1), cidx, kidx)
    out = _make_mlp_call()(pooled, W1, b1, W2, b2)
    return out.reshape(B, 1)


# speedup vs baseline: 23.1075x; 3.3034x over previous
import functools

import jax
import jax.numpy as jnp
from jax import lax
from jax.experimental import pallas as pl
from jax.experimental.pallas import tpu as pltpu
from jax.experimental.pallas import tpu_sc as plsc

B = 16384
L = 20
NUM_CAST = 1000
NUM_CREW = 1000
EMB = 32
HID = 128

NC = 2
NS = 16
LANES = 16
NW = NC * NS
BPW = B // NW


@functools.cache
def _make_pool_kernel(interpret=False):
    mesh = plsc.VectorSubcoreMesh(
        core_axis_name="c", subcore_axis_name="s",
        num_cores=NC, num_subcores=NS)

    @functools.partial(
        pl.kernel,
        out_type=jax.ShapeDtypeStruct((NW, BPW * 2 * EMB), jnp.float32),
        mesh=mesh,
        scratch_types=[
            pltpu.VMEM((NUM_CAST * EMB,), jnp.float32),
            pltpu.VMEM((NUM_CREW * EMB,), jnp.float32),
            pltpu.VMEM((BPW * L,), jnp.int32),
            pltpu.VMEM((BPW * L,), jnp.int32),
            pltpu.VMEM((BPW * 2 * EMB,), jnp.float32),
        ],
        compiler_params=pltpu.CompilerParams(needs_layout_passes=False),
        interpret=interpret,
    )
    def pool(cast_tab_hbm, crew_tab_hbm, cidx_hbm, kidx_hbm, out_hbm,
             cast_v, crew_v, cidx_v, kidx_v, out_v):
        wid = lax.axis_index("s") * NC + lax.axis_index("c")
        pltpu.sync_copy(cast_tab_hbm, cast_v)
        pltpu.sync_copy(crew_tab_hbm, crew_v)
        pltpu.sync_copy(cidx_hbm.at[wid], cidx_v)
        pltpu.sync_copy(kidx_hbm.at[wid], kidx_v)

        def row(b, carry):
            ib = b * L
            for idx_v, tab_v, off in ((cidx_v, cast_v, 0),
                                      (kidx_v, crew_v, EMB)):
                iv0 = idx_v[pl.ds(ib, LANES)] * EMB
                iv1 = idx_v[pl.ds(ib + L - LANES, LANES)] * EMB
                r0 = iv0[0]
                lo = tab_v[pl.ds(r0, LANES)]
                hi = tab_v[pl.ds(r0 + LANES, LANES)]
                for l in range(1, L):
                    r = iv0[l] if l < LANES else iv1[l - (L - LANES)]
                    lo = lo + tab_v[pl.ds(r, LANES)]
                    hi = hi + tab_v[pl.ds(r + LANES, LANES)]
                ob = b * (2 * EMB) + off
                out_v[pl.ds(ob, LANES)] = lo * (1.0 / L)
                out_v[pl.ds(ob + LANES, LANES)] = hi * (1.0 / L)
            return carry

        lax.fori_loop(0, BPW, row, 0)
        pltpu.sync_copy(out_v, out_hbm.at[wid])

    return pool


def _mlp_body(x_ref, w1_ref, b1_ref, w2_ref, b2_ref, o_ref):
    x = x_ref[...]
    h = lax.dot_general(x, w1_ref[...], (((1,), (0,)), ((), ())),
                        preferred_element_type=jnp.float32)
    h = jnp.maximum(h + b1_ref[...][None, :], 0.0)
    o = lax.dot_general(h, w2_ref[...], (((1,), (0,)), ((), ())),
                        preferred_element_type=jnp.float32)
    o_ref[...] = o + b2_ref[...][None, :]


@functools.cache
def _make_mlp_call(interpret=False):
    return pl.pallas_call(
        _mlp_body,
        grid=(NW,),
        in_specs=[
            pl.BlockSpec((BPW, 2 * EMB), lambda i: (i, 0)),
            pl.BlockSpec((2 * EMB, HID), lambda i: (0, 0)),
            pl.BlockSpec((HID,), lambda i: (0,)),
            pl.BlockSpec((HID, 1), lambda i: (0, 0)),
            pl.BlockSpec((1,), lambda i: (0,)),
        ],
        out_specs=pl.BlockSpec((BPW, 1), lambda i: (i, 0)),
        out_shape=jax.ShapeDtypeStruct((B, 1), jnp.float32),
        interpret=interpret,
    )


def kernel(cast_idx, crew_idx, cast_table, crew_table, W1, b1, W2, b2):
    cidx = cast_idx.astype(jnp.int32).reshape(NW, BPW * L)
    kidx = crew_idx.astype(jnp.int32).reshape(NW, BPW * L)
    pooled = _make_pool_kernel()(cast_table.reshape(-1),
                                 crew_table.reshape(-1), cidx, kidx)
    return _make_mlp_call()(pooled.reshape(B, 2 * EMB), W1, b1, W2, b2)


# speedup vs baseline: 24.7489x; 1.0710x over previous
import functools

import jax
import jax.numpy as jnp
import numpy as np
from jax import lax
from jax.experimental import pallas as pl
from jax.experimental.pallas import tpu as pltpu
from jax.experimental.pallas import tpu_sc as plsc

B = 16384
L = 20
NUM_CAST = 1000
NUM_CREW = 1000
EMB = 32
HID = 128

NC = 2
NS = 16
LANES = 16
NW = NC * NS
BPW = B // NW

_PERM = np.concatenate([np.arange(0, EMB, 2), np.arange(1, EMB, 2),
                        EMB + np.arange(0, EMB, 2), EMB + np.arange(1, EMB, 2)])


@functools.cache
def _make_pool_kernel(interpret=False):
    mesh = plsc.VectorSubcoreMesh(
        core_axis_name="c", subcore_axis_name="s",
        num_cores=NC, num_subcores=NS)

    @functools.partial(
        pl.kernel,
        out_type=jax.ShapeDtypeStruct((NW, BPW * 2 * EMB), jnp.float32),
        mesh=mesh,
        scratch_types=[
            pltpu.VMEM((NUM_CAST * EMB // 2,), jnp.int32),
            pltpu.VMEM((NUM_CREW * EMB // 2,), jnp.int32),
            pltpu.VMEM((BPW * L,), jnp.int32),
            pltpu.VMEM((BPW * L,), jnp.int32),
            pltpu.VMEM((BPW * 2 * EMB,), jnp.float32),
        ],
        compiler_params=pltpu.CompilerParams(needs_layout_passes=False),
        interpret=interpret,
    )
    def pool(cast_tab_hbm, crew_tab_hbm, cidx_hbm, kidx_hbm, out_hbm,
             cast_v, crew_v, cidx_v, kidx_v, out_v):
        wid = lax.axis_index("s") * NC + lax.axis_index("c")
        pltpu.sync_copy(cast_tab_hbm, cast_v)
        pltpu.sync_copy(crew_tab_hbm, crew_v)
        pltpu.sync_copy(cidx_hbm.at[wid], cidx_v)
        pltpu.sync_copy(kidx_hbm.at[wid], kidx_v)

        def row(b, carry):
            ib = b * L
            for idx_v, tab_v, off in ((cidx_v, cast_v, 0),
                                      (kidx_v, crew_v, EMB)):
                iv0 = idx_v[pl.ds(ib, LANES)] * (EMB // 2)
                iv1 = idx_v[pl.ds(ib + L - LANES, LANES)] * (EMB // 2)
                pa = [None, None, None, None]
                pb = [None, None, None, None]
                for l in range(L):
                    if l == 0:
                        r = iv0[0]
                    elif l < LANES:
                        r = iv0[l]
                    else:
                        r = iv1[l - (L - LANES)]
                    rw = plsc.bitcast(tab_v[pl.ds(r, LANES)], jnp.bfloat16)
                    a, c = plsc.unpack(rw, format=plsc.PackFormat.INTERLEAVED,
                                       preferred_element_type=jnp.float32)
                    k = l % 4
                    pa[k] = a if pa[k] is None else pa[k] + a
                    pb[k] = c if pb[k] is None else pb[k] + c
                ea = (pa[0] + pa[1]) + (pa[2] + pa[3])
                eb = (pb[0] + pb[1]) + (pb[2] + pb[3])
                ob = b * (2 * EMB) + off
                out_v[pl.ds(ob, LANES)] = ea
                out_v[pl.ds(ob + LANES, LANES)] = eb
            return carry

        lax.fori_loop(0, BPW, row, 0, unroll=8)
        pltpu.sync_copy(out_v, out_hbm.at[wid])

    return pool


def _mlp_body(x_ref, w1_ref, b1_ref, w2_ref, b2_ref, o_ref):
    x = x_ref[...]
    h = lax.dot_general(x, w1_ref[...], (((1,), (0,)), ((), ())),
                        preferred_element_type=jnp.float32)
    h = jnp.maximum(h + b1_ref[...][None, :], 0.0)
    o = lax.dot_general(h, w2_ref[...], (((1,), (0,)), ((), ())),
                        preferred_element_type=jnp.float32)
    o_ref[...] = o + b2_ref[...][None, :]


@functools.cache
def _make_mlp_call(interpret=False):
    return pl.pallas_call(
        _mlp_body,
        grid=(NW,),
        in_specs=[
            pl.BlockSpec((BPW, 2 * EMB), lambda i: (i, 0)),
            pl.BlockSpec((2 * EMB, HID), lambda i: (0, 0)),
            pl.BlockSpec((HID,), lambda i: (0,)),
            pl.BlockSpec((HID, 1), lambda i: (0, 0)),
            pl.BlockSpec((1,), lambda i: (0,)),
        ],
        out_specs=pl.BlockSpec((BPW, 1), lambda i: (i, 0)),
        out_shape=jax.ShapeDtypeStruct((B, 1), jnp.float32),
        interpret=interpret,
    )


def kernel(cast_idx, crew_idx, cast_table, crew_table, W1, b1, W2, b2):
    cidx = cast_idx.astype(jnp.int32).reshape(NW, BPW * L)
    kidx = crew_idx.astype(jnp.int32).reshape(NW, BPW * L)
    def _pack(tab, n):
        t = tab.astype(jnp.bfloat16).reshape(n, EMB // 2, 2)
        return lax.bitcast_convert_type(t, jnp.int32).reshape(-1)

    pooled = _make_pool_kernel()(
        _pack(cast_table, NUM_CAST), _pack(crew_table, NUM_CREW), cidx, kidx)
    w1p = W1[jnp.asarray(_PERM), :] * (1.0 / L)
    return _make_mlp_call()(pooled.reshape(B, 2 * EMB), w1p, b1, W2, b2)
